# baseline (speedup 1.0000x reference)
import functools

import jax
import jax.numpy as jnp
from jax import lax
from jax.experimental import pallas as pl
from jax.experimental.pallas import tpu as pltpu
from jax.experimental.pallas import tpu_sc as plsc

_NSEG = 512
_SEGCHUNK = 128
_NCHUNK = _NSEG // _SEGCHUNK
_ACC_ROWS = _NSEG + _SEGCHUNK
_NW = 32
_CH = 128


def _stage1(batch_ref, x_ref, w1_ref, b1_ref, w2_ref, w2row_ref, b2_ref,
            xe_ref, den_out_ref, den_ref, *, nblocks, bn):
    blk = pl.program_id(0)

    @pl.when(blk == 0)
    def _init():
        den_ref[...] = jnp.zeros_like(den_ref)

    xb = x_ref[...]
    xb16 = xb.astype(jnp.bfloat16)
    h = jnp.tanh(
        jax.lax.dot_general(xb16, w1_ref[...], (((1,), (0,)), ((), ())),
                            preferred_element_type=jnp.float32)
        + b1_ref[...])
    h16 = h.astype(jnp.bfloat16)
    lrow = jax.lax.dot_general(w2_ref[...], h16, (((0,), (1,)), ((), ())),
                               preferred_element_type=jnp.float32)
    erow = jnp.exp(lrow + b2_ref[0, 0]).astype(jnp.bfloat16)
    lcol = jnp.sum(h * w2row_ref[...], axis=1, keepdims=True)
    xe_ref[...] = xb * jnp.exp(lcol + b2_ref[0, 0])

    brow = batch_ref[0]
    bmin = batch_ref[0, 0, 0]
    bmax = batch_ref[0, 0, bn - 1]
    base = (bmin // 8) * 8
    ones8 = jnp.ones((bn, 8), jnp.bfloat16)
    subl = jax.lax.broadcasted_iota(jnp.int16, (_SEGCHUNK, bn), 0)

    def _den(anchor, sl):
        rel = (brow - anchor).astype(jnp.int16)
        owt = jnp.where(rel == subl, erow, jnp.bfloat16(0))
        dcol = jax.lax.dot_general(owt, ones8, (((1,), (0,)), ((), ())),
                                   preferred_element_type=jnp.float32)
        den_ref[sl, :] = den_ref[sl, :] + dcol[:, 0:1]

    @pl.when(bmax - base < _SEGCHUNK)
    def _fast():
        _den(base, pl.ds(base, _SEGCHUNK))

    @pl.when(bmax - base >= _SEGCHUNK)
    def _slow():
        for c in range(_NCHUNK):
            @pl.when((bmin < (c + 1) * _SEGCHUNK) & (bmax >= c * _SEGCHUNK))
            def _chunk(c=c):
                _den(c * _SEGCHUNK, pl.ds(c * _SEGCHUNK, _SEGCHUNK))

    @pl.when(blk == nblocks - 1)
    def _finish():
        den_out_ref[...] = den_ref[0:_NSEG, :]


def _sc_scatter(xe_hbm, batch_hbm, zeros_hbm, out_hbm, buf, idx_v, idx_rem,
                acc, *, n):
    cid = lax.axis_index("c")
    sid = lax.axis_index("s")
    wid = cid * (_NW // 2) + sid
    per_w = (n // (_NW * _CH)) * _CH
    start = wid * per_w

    @pl.when(sid == 0)
    def _zero():
        pltpu.sync_copy(zeros_hbm, acc)

    plsc.subcore_barrier()

    def body(k, _):
        off = start + k * _CH
        pltpu.sync_copy(xe_hbm.at[pl.ds(off, _CH)], buf)
        pltpu.sync_copy(batch_hbm.at[pl.ds(off, _CH)], idx_v)
        pltpu.sync_copy(buf, acc.at[idx_v], add=True)
        return 0

    lax.fori_loop(0, per_w // _CH, body, 0)

    tail_lo = _NW * per_w

    @pl.when(wid == _NW - 1)
    def _tail():
        ntail_full = (n - tail_lo) // _CH

        def tbody(k, _):
            off = tail_lo + k * _CH
            pltpu.sync_copy(xe_hbm.at[pl.ds(off, _CH)], buf)
            pltpu.sync_copy(batch_hbm.at[pl.ds(off, _CH)], idx_v)
            pltpu.sync_copy(buf, acc.at[idx_v], add=True)
            return 0

        lax.fori_loop(0, ntail_full, tbody, 0)
        rem = n - tail_lo - ntail_full * _CH
        if rem:
            off = tail_lo + ntail_full * _CH
            pltpu.sync_copy(xe_hbm.at[pl.ds(off, rem)],
                            buf.at[pl.ds(0, rem)])
            pltpu.sync_copy(batch_hbm.at[pl.ds(off, rem)], idx_rem)
            pltpu.sync_copy(buf.at[pl.ds(0, rem)], acc.at[idx_rem],
                            add=True)

    plsc.subcore_barrier()

    @pl.when(sid == 0)
    def _flush():
        pltpu.sync_copy(acc, out_hbm.at[cid])


def _stage3(p_ref, den_ref, out_ref):
    acc = p_ref[0] + p_ref[1]
    out_ref[...] = acc / (den_ref[...] + 1e-16)


def kernel(x, batch, W1, b1, W2, b2):
    n, d = x.shape
    bn = 4000
    nblocks = pl.cdiv(n, bn)

    batch3d = batch.reshape(nblocks, 1, bn)
    b1r = b1.reshape(1, d)
    w2col = W2.astype(jnp.bfloat16)
    w2row = W2.reshape(1, d)
    b2r = b2.reshape(1, 1)
    w1_16 = W1.astype(jnp.bfloat16)

    xe, den = pl.pallas_call(
        functools.partial(_stage1, nblocks=nblocks, bn=bn),
        grid=(nblocks,),
        in_specs=[
            pl.BlockSpec((1, 1, bn), lambda i: (i, 0, 0)),
            pl.BlockSpec((bn, d), lambda i: (i, 0)),
            pl.BlockSpec((d, d), lambda i: (0, 0)),
            pl.BlockSpec((1, d), lambda i: (0, 0)),
            pl.BlockSpec((d, 1), lambda i: (0, 0)),
            pl.BlockSpec((1, d), lambda i: (0, 0)),
            pl.BlockSpec((1, 1), lambda i: (0, 0)),
        ],
        out_specs=[
            pl.BlockSpec((bn, d), lambda i: (i, 0)),
            pl.BlockSpec((_NSEG, 1), lambda i: (0, 0)),
        ],
        out_shape=[
            jax.ShapeDtypeStruct((n, d), jnp.float32),
            jax.ShapeDtypeStruct((_NSEG, 1), jnp.float32),
        ],
        scratch_shapes=[pltpu.VMEM((_ACC_ROWS, 1), jnp.float32)],
        compiler_params=pltpu.CompilerParams(
            dimension_semantics=("arbitrary",),
        ),
    )(batch3d, x, w1_16, b1r, w2col, w2row, b2r)

    zeros = jnp.zeros((_NSEG, d), jnp.float32)
    mesh = plsc.VectorSubcoreMesh(core_axis_name="c", subcore_axis_name="s")
    partials = pl.kernel(
        functools.partial(_sc_scatter, n=n),
        mesh=mesh,
        out_type=jax.ShapeDtypeStruct((2, _NSEG, d), jnp.float32),
        scratch_types=[
            pltpu.VMEM((_CH, d), jnp.float32),
            pltpu.VMEM((_CH,), jnp.int32),
            pltpu.VMEM((32,), jnp.int32),
            pltpu.VMEM_SHARED((_NSEG, d), jnp.float32),
        ],
    )(xe, batch, zeros)

    out = pl.pallas_call(
        _stage3,
        out_shape=jax.ShapeDtypeStruct((_NSEG, d), jnp.float32),
    )(partials, den)
    return out


# speedup vs baseline: 3.3471x; 3.3471x over previous
import functools

import jax
import jax.numpy as jnp
from jax.experimental import pallas as pl
from jax.experimental.pallas import tpu as pltpu

_NSEG = 512
_SEGCHUNK = 128
_NCHUNK = _NSEG // _SEGCHUNK
_ACC_ROWS = _NSEG + _SEGCHUNK


def _body(batch_ref, x_ref, w1_ref, b1_ref, w2_ref, b2_ref, out_ref,
          acc_ref, den_ref, *, nblocks, bn):
    blk = pl.program_id(0)

    @pl.when(blk == 0)
    def _init():
        acc_ref[...] = jnp.zeros_like(acc_ref)
        den_ref[...] = jnp.zeros_like(den_ref)

    xb16 = x_ref[...].astype(jnp.bfloat16)
    h = jnp.tanh(
        jax.lax.dot_general(xb16, w1_ref[...], (((1,), (0,)), ((), ())),
                            preferred_element_type=jnp.float32)
        + b1_ref[...])
    lrow = jax.lax.dot_general(w2_ref[...], h.astype(jnp.bfloat16),
                               (((0,), (1,)), ((), ())),
                               preferred_element_type=jnp.float32)
    erow = jnp.exp(lrow + b2_ref[0, 0]).astype(jnp.bfloat16)

    brow = batch_ref[0]
    bmin = batch_ref[0, 0, 0]
    bmax = batch_ref[0, 0, bn - 1]
    base = (bmin // 8) * 8
    ones8 = jnp.ones((bn, 8), jnp.bfloat16)
    subl = jax.lax.broadcasted_iota(jnp.int16, (_SEGCHUNK, bn), 0)

    def _scatter(anchor, sl):
        rel = (brow - anchor).astype(jnp.int16)
        owt = jnp.where(rel == subl, erow, jnp.bfloat16(0))
        num = jax.lax.dot_general(owt, xb16, (((1,), (0,)), ((), ())),
                                  preferred_element_type=jnp.float32)
        dcol = jax.lax.dot_general(owt, ones8, (((1,), (0,)), ((), ())),
                                   preferred_element_type=jnp.float32)
        acc_ref[sl, :] = acc_ref[sl, :] + num
        den_ref[sl, :] = den_ref[sl, :] + dcol[:, 0:1]

    @pl.when(bmax - base < _SEGCHUNK)
    def _fast():
        _scatter(base, pl.ds(base, _SEGCHUNK))

    @pl.when(bmax - base >= _SEGCHUNK)
    def _slow():
        for c in range(_NCHUNK):
            @pl.when((bmin < (c + 1) * _SEGCHUNK) & (bmax >= c * _SEGCHUNK))
            def _chunk(c=c):
                _scatter(c * _SEGCHUNK, pl.ds(c * _SEGCHUNK, _SEGCHUNK))

    @pl.when(blk == nblocks - 1)
    def _finish():
        out_ref[...] = acc_ref[0:_NSEG, :] / (den_ref[0:_NSEG, :] + 1e-16)


def kernel(x, batch, W1, b1, W2, b2):
    n, d = x.shape
    bn = 4000
    nblocks = pl.cdiv(n, bn)

    batch3d = batch.reshape(nblocks, 1, bn)
    b1r = b1.reshape(1, d)
    w2col = W2.astype(jnp.bfloat16)
    b2r = b2.reshape(1, 1)
    w1_16 = W1.astype(jnp.bfloat16)

    out = pl.pallas_call(
        functools.partial(_body, nblocks=nblocks, bn=bn),
        grid=(nblocks,),
        in_specs=[
            pl.BlockSpec((1, 1, bn), lambda i: (i, 0, 0)),
            pl.BlockSpec((bn, d), lambda i: (i, 0)),
            pl.BlockSpec((d, d), lambda i: (0, 0)),
            pl.BlockSpec((1, d), lambda i: (0, 0)),
            pl.BlockSpec((d, 1), lambda i: (0, 0)),
            pl.BlockSpec((1, 1), lambda i: (0, 0)),
        ],
        out_specs=pl.BlockSpec((_NSEG, d), lambda i: (0, 0)),
        out_shape=jax.ShapeDtypeStruct((_NSEG, d), jnp.float32),
        scratch_shapes=[
            pltpu.VMEM((_ACC_ROWS, d), jnp.float32),
            pltpu.VMEM((_ACC_ROWS, 1), jnp.float32),
        ],
        compiler_params=pltpu.CompilerParams(
            dimension_semantics=("arbitrary",),
        ),
    )(batch3d, x, w1_16, b1r, w2col, b2r)
    return out
